# baseline (speedup 1.0000x reference)
import functools

import jax
import jax.numpy as jnp
from jax.experimental import pallas as pl

_KEY_HI = 0
_KEY_LO = 42
_INV_KPROB = 2.0

_BLOCK = 131072

_ROT_A = (13, 15, 26, 6)
_ROT_B = (17, 29, 16, 24)


def _dropout_block(values_ref, out_ref):
    j = pl.program_id(0)
    base = (j * _BLOCK).astype(jnp.uint32)
    idx = jax.lax.broadcasted_iota(jnp.uint32, values_ref.shape, 1) + base

    ks0 = jnp.uint32(_KEY_HI)
    ks1 = jnp.uint32(_KEY_LO)
    ks2 = jnp.uint32(_KEY_HI ^ _KEY_LO ^ 0x1BD11BDA)

    x0 = jnp.full(values_ref.shape, ks0, jnp.uint32)
    x1 = idx + ks1

    def rotl(x, d):
        return (x << jnp.uint32(d)) | (x >> jnp.uint32(32 - d))

    injections = ((ks1, ks2), (ks2, ks0), (ks0, ks1), (ks1, ks2), (ks2, ks0))
    for i, (a, b) in enumerate(injections):
        for r in _ROT_A if i % 2 == 0 else _ROT_B:
            x0 = x0 + x1
            x1 = rotl(x1, r)
            x1 = x1 ^ x0
        x0 = x0 + a
        x1 = x1 + b + jnp.uint32(i + 1)

    bits = x0 ^ x1
    keep = (bits >> jnp.uint32(31)) == jnp.uint32(1)
    v = values_ref[...]
    out_ref[...] = jnp.where(keep, v * _INV_KPROB, 0.0)


@functools.partial(jax.jit, static_argnames=("interpret",))
def _sparse_dropout(values, interpret=False):
    nnz = values.shape[0]
    grid = pl.cdiv(nnz, _BLOCK)
    out = pl.pallas_call(
        _dropout_block,
        grid=(grid,),
        in_specs=[pl.BlockSpec((1, _BLOCK), lambda j: (0, j))],
        out_specs=pl.BlockSpec((1, _BLOCK), lambda j: (0, j)),
        out_shape=jax.ShapeDtypeStruct((1, nnz), values.dtype),
        interpret=interpret,
    )(values.reshape(1, nnz))
    return out.reshape(nnz)


def kernel(indices, values):
    return indices, _sparse_dropout(values)


# speedup vs baseline: 2.1919x; 2.1919x over previous
import functools

import jax
import jax.numpy as jnp
from jax.experimental import pallas as pl

_KEY_HI = 0
_KEY_LO = 42
_INV_KPROB = 2.0

_BLOCK = 32768
_CHUNK = 4096

_ROT_A = (13, 15, 26, 6)
_ROT_B = (17, 29, 16, 24)


def _threefry_keep(idx):
    ks0 = jnp.uint32(_KEY_HI)
    ks1 = jnp.uint32(_KEY_LO)
    ks2 = jnp.uint32(_KEY_HI ^ _KEY_LO ^ 0x1BD11BDA)

    x0 = jnp.full(idx.shape, ks0, jnp.uint32)
    x1 = idx + ks1

    def rotl(x, d):
        return (x << jnp.uint32(d)) | (x >> jnp.uint32(32 - d))

    injections = ((ks1, ks2), (ks2, ks0), (ks0, ks1), (ks1, ks2), (ks2, ks0))
    for i, (a, b) in enumerate(injections):
        for r in _ROT_A if i % 2 == 0 else _ROT_B:
            x0 = x0 + x1
            x1 = rotl(x1, r)
            x1 = x1 ^ x0
        x0 = x0 + a
        x1 = x1 + b + jnp.uint32(i + 1)

    bits = x0 ^ x1
    return (bits >> jnp.uint32(31)) == jnp.uint32(1)


def _dropout_block(values_ref, out_ref):
    j = pl.program_id(0)
    rows = _CHUNK // 8
    for c in range(_BLOCK // _CHUNK):
        base = (j * _BLOCK + c * _CHUNK).astype(jnp.uint32)
        idx = (
            base
            + jax.lax.broadcasted_iota(jnp.uint32, (8, rows), 0) * rows
            + jax.lax.broadcasted_iota(jnp.uint32, (8, rows), 1)
        )
        keep = _threefry_keep(idx)
        v = values_ref[0:1, c * _CHUNK : (c + 1) * _CHUNK].reshape(8, rows)
        out = jnp.where(keep, v * _INV_KPROB, 0.0)
        out_ref[0:1, c * _CHUNK : (c + 1) * _CHUNK] = out.reshape(1, _CHUNK)


@functools.partial(jax.jit, static_argnames=("interpret",))
def _sparse_dropout(values, interpret=False):
    nnz = values.shape[0]
    grid = pl.cdiv(nnz, _BLOCK)
    out = pl.pallas_call(
        _dropout_block,
        grid=(grid,),
        in_specs=[pl.BlockSpec((1, _BLOCK), lambda j: (0, j))],
        out_specs=pl.BlockSpec((1, _BLOCK), lambda j: (0, j)),
        out_shape=jax.ShapeDtypeStruct((1, nnz), values.dtype),
        interpret=interpret,
    )(values.reshape(1, nnz))
    return out.reshape(nnz)


def kernel(indices, values):
    return indices, _sparse_dropout(values)


# speedup vs baseline: 6.5343x; 2.9811x over previous
import functools

import jax
import jax.numpy as jnp
from jax.experimental import pallas as pl

_KEY_HI = 0
_KEY_LO = 42
_INV_KPROB = 2.0

_BLOCK = 32768
_CHUNK = 4096

_ROT_A = (13, 15, 26, 6)
_ROT_B = (17, 29, 16, 24)


def _threefry_keep(idx):
    ks0 = jnp.uint32(_KEY_HI)
    ks1 = jnp.uint32(_KEY_LO)
    ks2 = jnp.uint32(_KEY_HI ^ _KEY_LO ^ 0x1BD11BDA)

    x0 = jnp.full(idx.shape, ks0, jnp.uint32)
    x1 = idx + ks1

    def rotl(x, d):
        return (x << jnp.uint32(d)) | (x >> jnp.uint32(32 - d))

    injections = ((ks1, ks2), (ks2, ks0), (ks0, ks1), (ks1, ks2), (ks2, ks0))
    for i, (a, b) in enumerate(injections):
        for r in _ROT_A if i % 2 == 0 else _ROT_B:
            x0 = x0 + x1
            x1 = rotl(x1, r)
            x1 = x1 ^ x0
        x0 = x0 + a
        x1 = x1 + b + jnp.uint32(i + 1)

    bits = x0 ^ x1
    return (bits >> jnp.uint32(31)) == jnp.uint32(1)


def _dropout_block(values_ref, out_ref):
    j = pl.program_id(0)
    rows = _CHUNK // 8
    for c in range(_BLOCK // _CHUNK):
        base = (j * _BLOCK + c * _CHUNK).astype(jnp.uint32)
        idx = (
            base
            + jax.lax.broadcasted_iota(jnp.uint32, (8, rows), 0) * rows
            + jax.lax.broadcasted_iota(jnp.uint32, (8, rows), 1)
        )
        keep = _threefry_keep(idx)
        v = values_ref[c * _CHUNK : (c + 1) * _CHUNK].reshape(8, rows)
        out = jnp.where(keep, v * _INV_KPROB, 0.0)
        out_ref[c * _CHUNK : (c + 1) * _CHUNK] = out.reshape(_CHUNK)


@functools.partial(jax.jit, static_argnames=("interpret",))
def _sparse_dropout(values, interpret=False):
    nnz = values.shape[0]
    grid = pl.cdiv(nnz, _BLOCK)
    return pl.pallas_call(
        _dropout_block,
        grid=(grid,),
        in_specs=[pl.BlockSpec((_BLOCK,), lambda j: (j,))],
        out_specs=pl.BlockSpec((_BLOCK,), lambda j: (j,)),
        out_shape=jax.ShapeDtypeStruct((nnz,), values.dtype),
        interpret=interpret,
    )(values)


def kernel(indices, values):
    return indices, _sparse_dropout(values)
